# baseline (speedup 1.0000x reference)
import functools

import jax
import jax.numpy as jnp
from jax import lax
from jax.experimental import pallas as pl
from jax.experimental.pallas import tpu as pltpu
from jax.experimental.pallas import tpu_sc as plsc

K = 16
N = 1024
BIG_IDX = 2048.0
INF = float("inf")
NCH = 4


def _knn_body(cur0_ref, pts_ref, outx_ref, dist_ref, idx_ref, cur_ref):
    i = pl.program_id(1)

    @pl.when(i == 0)
    def _init():
        cur_ref[...] = cur0_ref[0]

    kT = jnp.transpose(pts_ref[0, 0], (1, 0))
    kx = kT[0:1, :]
    ky = kT[1:2, :]
    kz = kT[2:3, :]
    qx = cur_ref[:, 0:1]
    qy = cur_ref[:, 1:2]
    qz = cur_ref[:, 2:3]

    dx = qx - kx
    dy = qy - ky
    dz = qz - kz
    d = dx * dx + dy * dy
    d = d + dz * dz

    iota = jax.lax.broadcasted_iota(jnp.int32, (1, N), 1).astype(jnp.float32)

    for j in range(K):
        m = jnp.min(d, axis=1, keepdims=True)
        cand = jnp.where(d == m, iota, BIG_IDX)
        a = jnp.min(cand, axis=1, keepdims=True)
        dist_ref[0, 0, :, j : j + 1] = m
        idx_ref[0, 0, :, j : j + 1] = a.astype(jnp.int32)
        onehot = cand == a
        if j == 0:
            xc = jnp.min(jnp.where(onehot, kx, INF), axis=1, keepdims=True)
            yc = jnp.min(jnp.where(onehot, ky, INF), axis=1, keepdims=True)
            zc = jnp.min(jnp.where(onehot, kz, INF), axis=1, keepdims=True)
            cur_ref[:, 0:1] = xc
            cur_ref[:, 1:2] = yc
            cur_ref[:, 2:3] = zc
            outx_ref[0, 0, :, 0:1] = xc
            outx_ref[0, 0, :, 1:2] = yc
            outx_ref[0, 0, :, 2:3] = zc
        if j < K - 1:
            d = jnp.where(onehot, INF, d)


def _knn_topk(cur0, pts_chunk):
    b, tc, n, c = pts_chunk.shape
    grid = (b, tc)
    out_shapes = (
        jax.ShapeDtypeStruct((b, tc, n, c), jnp.float32),
        jax.ShapeDtypeStruct((b, tc, n, K), jnp.float32),
        jax.ShapeDtypeStruct((b, tc, n, K), jnp.int32),
    )
    return pl.pallas_call(
        _knn_body,
        grid=grid,
        in_specs=[
            pl.BlockSpec((1, n, c), lambda bi, ti: (bi, 0, 0)),
            pl.BlockSpec((1, 1, n, c), lambda bi, ti: (bi, ti, 0, 0)),
        ],
        out_specs=(
            pl.BlockSpec((1, 1, n, c), lambda bi, ti: (bi, ti, 0, 0)),
            pl.BlockSpec((1, 1, n, K), lambda bi, ti: (bi, ti, 0, 0)),
            pl.BlockSpec((1, 1, n, K), lambda bi, ti: (bi, ti, 0, 0)),
        ),
        out_shape=out_shapes,
        scratch_shapes=[pltpu.VMEM((n, c), jnp.float32)],
        compiler_params=pltpu.CompilerParams(
            dimension_semantics=("arbitrary", "arbitrary"),
        ),
    )(cur0, pts_chunk)


def _sc_gather(pts_flat, idx_chunk_flat, b, t, tc, t0, n):
    npairs = b * tc
    pairs_per_w = npairs // 32
    mesh = plsc.VectorSubcoreMesh(core_axis_name="c", subcore_axis_name="s")
    row_out = n * K * 3

    @functools.partial(
        pl.kernel,
        mesh=mesh,
        out_type=(
            jax.ShapeDtypeStruct((npairs, row_out), jnp.float32),
            jax.ShapeDtypeStruct((npairs, row_out), jnp.float32),
        ),
        scratch_types=[
            pltpu.VMEM((3 * n,), jnp.float32),
            pltpu.VMEM((3 * n,), jnp.float32),
            pltpu.VMEM((n * K,), jnp.int32),
            pltpu.VMEM((row_out,), jnp.float32),
            pltpu.VMEM((row_out,), jnp.float32),
        ],
        compiler_params=pltpu.CompilerParams(needs_layout_passes=False),
    )
    def sc_kern(pts_hbm, idx_hbm, pp_hbm, pf_hbm, cur_v, prev_v, idx_v, pp_v, pf_v):
        wid = lax.axis_index("s") * 2 + lax.axis_index("c")
        jota3 = lax.iota(jnp.int32, 16) * 3

        for p in range(pairs_per_w):
            lp = wid * pairs_per_w + p
            bl = lax.div(lp, tc)
            tg = t0 + lax.rem(lp, tc)
            g = bl * t + tg
            g_prev = jnp.where(tg == 0, g, g - 1)
            pltpu.sync_copy(pts_hbm.at[g], cur_v)
            pltpu.sync_copy(pts_hbm.at[g_prev], prev_v)
            pltpu.sync_copy(idx_hbm.at[lp], idx_v)

            def q_body(q, _):
                iv0 = idx_v[pl.ds(q * K, 16)]
                iv3 = iv0 * 3
                qbase = q * (K * 3)
                for ci in range(3):
                    iv = iv3 + ci if ci else iv3
                    sidx = jota3 + (qbase + ci)
                    vals = plsc.load_gather(cur_v, [iv])
                    plsc.store_scatter(pp_v, [sidx], vals)
                    fvals = plsc.load_gather(prev_v, [iv])
                    plsc.store_scatter(pf_v, [sidx], fvals)
                return _

            lax.fori_loop(0, n, q_body, None)
            pltpu.sync_copy(pp_v, pp_hbm.at[lp])
            pltpu.sync_copy(pf_v, pf_hbm.at[lp])

    return sc_kern(pts_flat, idx_chunk_flat)


def kernel(point_seq):
    b, t, n, c = point_seq.shape
    tc = t // NCH
    pts_flat = point_seq.reshape(b * t, n * c)

    cur0 = point_seq[:, 0]
    outx_l, dist_l, idx_l, pp_l, pf_l = [], [], [], [], []
    for ch in range(NCH):
        t0 = ch * tc
        pts_chunk = point_seq[:, t0 : t0 + tc]
        out_x, distances, idxs = _knn_topk(cur0, pts_chunk)
        cur0 = out_x[:, -1]
        idx_flat = idxs.reshape(b * tc, n * K)
        pp, pf = _sc_gather(pts_flat, idx_flat, b, t, tc, t0, n)
        outx_l.append(out_x)
        dist_l.append(distances)
        idx_l.append(idxs)
        pp_l.append(pp.reshape(b, tc, n, K, c))
        pf_l.append(pf.reshape(b, tc, n, K, c))

    out_x = jnp.concatenate(outx_l, axis=1)
    distances = jnp.concatenate(dist_l, axis=1)
    idxs = jnp.concatenate(idx_l, axis=1)
    patchlet_points = jnp.concatenate(pp_l, axis=1)
    patchlet_feats = jnp.concatenate(pf_l, axis=1)
    return out_x, distances, idxs, patchlet_points, patchlet_feats


# speedup vs baseline: 1.1902x; 1.1902x over previous
import functools

import jax
import jax.numpy as jnp
from jax import lax
from jax.experimental import pallas as pl
from jax.experimental.pallas import tpu as pltpu
from jax.experimental.pallas import tpu_sc as plsc

K = 16
N = 1024
BIG_IDX = 2048.0
INF = float("inf")


def _knn_body(pts_ref, outx_ref, dist_ref, idx_ref, cur_ref):
    i = pl.program_id(1)

    @pl.when(i == 0)
    def _init():
        cur_ref[...] = pts_ref[0, 0]

    kT = jnp.transpose(pts_ref[0, 0], (1, 0))
    kx = kT[0:1, :]
    ky = kT[1:2, :]
    kz = kT[2:3, :]
    qx = cur_ref[:, 0:1]
    qy = cur_ref[:, 1:2]
    qz = cur_ref[:, 2:3]

    dx = qx - kx
    dy = qy - ky
    dz = qz - kz
    d = dx * dx + dy * dy
    d = d + dz * dz

    iota = jax.lax.broadcasted_iota(jnp.int32, (1, N), 1).astype(jnp.float32)

    for j in range(K):
        m = jnp.min(d, axis=1, keepdims=True)
        cand = jnp.where(d == m, iota, BIG_IDX)
        a = jnp.min(cand, axis=1, keepdims=True)
        dist_ref[0, 0, :, j : j + 1] = m
        idx_ref[0, 0, :, j : j + 1] = a.astype(jnp.int32)
        onehot = cand == a
        if j == 0:
            onehot_f = jnp.where(onehot, 1.0, 0.0)
            xc3 = jax.lax.dot_general(
                onehot_f,
                pts_ref[0, 0],
                (((1,), (0,)), ((), ())),
                preferred_element_type=jnp.float32,
            )
            cur_ref[...] = xc3
            outx_ref[0, 0] = xc3
        if j < K - 1:
            d = jnp.where(onehot, INF, d)


def _knn_topk(point_seq):
    b, t, n, c = point_seq.shape
    grid = (b, t)
    out_shapes = (
        jax.ShapeDtypeStruct((b, t, n, c), jnp.float32),
        jax.ShapeDtypeStruct((b, t, n, K), jnp.float32),
        jax.ShapeDtypeStruct((b, t, n, K), jnp.int32),
    )
    return pl.pallas_call(
        _knn_body,
        grid=grid,
        in_specs=[
            pl.BlockSpec((1, 1, n, c), lambda bi, ti: (bi, ti, 0, 0)),
        ],
        out_specs=(
            pl.BlockSpec((1, 1, n, c), lambda bi, ti: (bi, ti, 0, 0)),
            pl.BlockSpec((1, 1, n, K), lambda bi, ti: (bi, ti, 0, 0)),
            pl.BlockSpec((1, 1, n, K), lambda bi, ti: (bi, ti, 0, 0)),
        ),
        out_shape=out_shapes,
        scratch_shapes=[pltpu.VMEM((n, c), jnp.float32)],
        compiler_params=pltpu.CompilerParams(
            dimension_semantics=("arbitrary", "arbitrary"),
        ),
    )(point_seq)


def _sc_gather(pts_flat, idx_flat, b, t, n):
    npairs = b * t
    pairs_per_w = npairs // 32
    mesh = plsc.VectorSubcoreMesh(core_axis_name="c", subcore_axis_name="s")
    row_out = n * K * 3

    @functools.partial(
        pl.kernel,
        mesh=mesh,
        out_type=(
            jax.ShapeDtypeStruct((npairs, row_out), jnp.float32),
            jax.ShapeDtypeStruct((npairs, row_out), jnp.float32),
        ),
        scratch_types=[
            pltpu.VMEM((3 * n,), jnp.float32),
            pltpu.VMEM((3 * n,), jnp.float32),
            pltpu.VMEM((n * K,), jnp.int32),
            pltpu.VMEM((row_out,), jnp.float32),
            pltpu.VMEM((row_out,), jnp.float32),
        ],
        compiler_params=pltpu.CompilerParams(needs_layout_passes=False),
    )
    def sc_kern(pts_hbm, idx_hbm, pp_hbm, pf_hbm, cur_v, prev_v, idx_v, pp_v, pf_v):
        wid = lax.axis_index("s") * 2 + lax.axis_index("c")
        jota3 = lax.iota(jnp.int32, 16) * 3

        for p in range(pairs_per_w):
            pair = wid * pairs_per_w + p
            ti = lax.rem(pair, t)
            prev_pair = jnp.where(ti == 0, pair, pair - 1)
            pltpu.sync_copy(pts_hbm.at[pair], cur_v)
            pltpu.sync_copy(pts_hbm.at[prev_pair], prev_v)
            pltpu.sync_copy(idx_hbm.at[pair], idx_v)

            def q_body(q, _):
                iv0 = idx_v[pl.ds(q * K, 16)]
                iv3 = iv0 * 3
                qbase = q * (K * 3)
                for ci in range(3):
                    iv = iv3 + ci if ci else iv3
                    sidx = jota3 + (qbase + ci)
                    vals = plsc.load_gather(cur_v, [iv])
                    plsc.store_scatter(pp_v, [sidx], vals)
                    fvals = plsc.load_gather(prev_v, [iv])
                    plsc.store_scatter(pf_v, [sidx], fvals)
                return _

            lax.fori_loop(0, n, q_body, None)
            pltpu.sync_copy(pp_v, pp_hbm.at[pair])
            pltpu.sync_copy(pf_v, pf_hbm.at[pair])

    return sc_kern(pts_flat, idx_flat)


def kernel(point_seq):
    b, t, n, c = point_seq.shape
    out_x, distances, idxs = _knn_topk(point_seq)

    pts_flat = point_seq.reshape(b * t, n * c)
    idx_flat = idxs.reshape(b * t, n * K)
    pp, pf = _sc_gather(pts_flat, idx_flat, b, t, n)
    patchlet_points = pp.reshape(b, t, n, K, c)
    patchlet_feats = pf.reshape(b, t, n, K, c)
    return out_x, distances, idxs, patchlet_points, patchlet_feats
